# Initial kernel scaffold; baseline (speedup 1.0000x reference)
import jax
import jax.numpy as jnp
from jax import lax
from jax.experimental import pallas as pl
from jax.experimental.pallas import tpu as pltpu
from jax.experimental.pallas import tpu_sc as plsc

NUSER = 50000
NLOC = 50000
BB = 4
E2 = 100000
D = 200
EP = 100352
TPT1 = EP // 32
SLICE2 = EP // 16
NV2 = SLICE2 // 16
HALF = NUSER // 2
NCH = 4
CS = HALF // NCH
CSP = 6272
DUMMY = 6256
RZ = CSP // 16
MR = 391
G = 128
CLCAP = SLICE2 + G
SLOT = 400
ELCAP = 416
BIGI = jnp.int32(1 << 30)
F32 = jnp.float32
I32 = jnp.int32


def _mesh():
    return plsc.VectorSubcoreMesh(
        core_axis_name="c", subcore_axis_name="s", num_cores=2, num_subcores=16
    )


def _iota16():
    return lax.iota(I32, 16)


def _run_k1(ul, ll, su, sl):
    @pl.kernel(
        out_type=(
            jax.ShapeDtypeStruct((BB, EP), I32),
            jax.ShapeDtypeStruct((BB, EP), I32),
        ),
        mesh=_mesh(),
        scratch_types=[
            pltpu.VMEM((NUSER,), I32),
            pltpu.VMEM((TPT1,), I32),
            pltpu.VMEM((TPT1,), I32),
        ],
    )
    def k1(ul_h, ll_h, su_h, sl_h, uidx_h, lidx_h, tbl, ibuf, obuf):
        cid = lax.axis_index("c")
        sid = lax.axis_index("s")
        wid = sid * 2 + cid
        base_e = wid * TPT1
        iota = _iota16()
        for tbl_h, lnk_h, out_h in ((su_h, ul_h, uidx_h), (sl_h, ll_h, lidx_h)):
            pltpu.sync_copy(tbl_h, tbl)
            for b in range(BB):
                pltpu.sync_copy(lnk_h.at[b, pl.ds(base_e, TPT1)], ibuf)

                def body(i, _):
                    iv = ibuf[pl.ds(i * 16, 16)]
                    vals = plsc.load_gather(tbl, [iv])
                    gpos = base_e + i * 16 + iota
                    obuf[pl.ds(i * 16, 16)] = jnp.where(gpos < E2, vals, BIGI)
                    return 0

                lax.fori_loop(0, TPT1 // 16, body, 0)
                pltpu.sync_copy(obuf, out_h.at[b, pl.ds(base_e, TPT1)])

    return k1(ul, ll, su, sl)


def _run_k2(xloc, uidx, lidx):
    @pl.kernel(
        out_type=(
            jax.ShapeDtypeStruct((BB, NUSER, D), F32),
            jax.ShapeDtypeStruct((BB, 2, NCH, 16, SLOT), I32),
            jax.ShapeDtypeStruct((BB, 2, NCH, 16, 8), I32),
            jax.ShapeDtypeStruct((BB, 2, 16, 208), F32),
        ),
        mesh=_mesh(),
        scratch_types=[
            pltpu.VMEM((SLICE2,), I32),
            pltpu.VMEM((SLICE2,), I32),
            pltpu.VMEM((CLCAP,), I32),
            pltpu.VMEM((CLCAP,), I32),
            pltpu.VMEM((G,), I32),
            pltpu.VMEM((G,), I32),
            pltpu.VMEM((G, D), F32),
            pltpu.VMEM((G,), F32),
            pltpu.VMEM((G, D), F32),
            pltpu.VMEM((RZ + 8,), F32),
            pltpu.VMEM((CSP,), F32),
            pltpu.VMEM((ELCAP,), I32),
            pltpu.VMEM((16,), I32),
            pltpu.VMEM((208,), F32),
            pltpu.VMEM_SHARED((CSP, D), F32),
            pltpu.VMEM_SHARED((CSP,), F32),
            pltpu.SemaphoreType.DMA,
        ],
    )
    def k2(xloc_h, uidx_h, lidx_h, out_h, slots_h, lens_h, parts_h,
           u_sl, l_sl, cl_cid, cl_loc, st_cid, st_loc, gbuf, ones_b,
           zbuf, zrow, cnt_t, elist, lbuf, acc, sums_sh, cnt_sh, sem):
        cid = lax.axis_index("c")
        sid = lax.axis_index("s")
        iota = _iota16()
        half_base = cid * HALF
        zero16 = jnp.zeros((16,), F32)

        def zr_init(i, _):
            for w in range(12):
                zbuf[i, pl.ds(w * 16, 16)] = zero16
            zbuf[i, pl.ds(D - 16, 16)] = zero16
            return 0

        lax.fori_loop(0, G, zr_init, 0)
        for q in range(G // 16):
            ones_b[pl.ds(q * 16, 16)] = jnp.full((16,), 1.0, F32)
        for q in range((RZ + 8) // 16):
            zrow[pl.ds(q * 16, 16)] = zero16

        for b in range(BB):
            ebase = sid * SLICE2
            pltpu.sync_copy(uidx_h.at[b, pl.ds(ebase, SLICE2)], u_sl)
            pltpu.sync_copy(lidx_h.at[b, pl.ds(ebase, SLICE2)], l_sl)
            for w in range(13):
                acc[pl.ds(w * 16, 16)] = zero16

            for k in range(NCH):
                chunk_base = half_base + k * CS
                plsc.subcore_barrier()
                rz = sid * RZ
                for (o, n) in ((0, 128), (128, 128), (256, 128), (384, 8)):
                    pltpu.sync_copy(zbuf.at[pl.ds(0, n)],
                                    sums_sh.at[pl.ds(rz + o, n)])
                pltpu.sync_copy(zrow.at[pl.ds(0, RZ)],
                                cnt_sh.at[pl.ds(rz, RZ)])
                plsc.subcore_barrier()

                def cbody(i, off):
                    uv = u_sl[pl.ds(i * 16, 16)]
                    cv = uv - chunk_base
                    m = (cv >= 0) & (cv < CS)
                    lv = l_sl[pl.ds(i * 16, 16)]
                    plsc.store_compressed(cl_cid.at[pl.ds(off, 16)], cv, mask=m)
                    plsc.store_compressed(cl_loc.at[pl.ds(off, 16)], lv, mask=m)
                    return off + jnp.sum(m.astype(I32))

                ncg = lax.fori_loop(0, NV2, cbody, jnp.int32(0))
                for t in range(G // 16):
                    cl_cid[pl.ds(ncg + t * 16, 16)] = jnp.full((16,), DUMMY, I32)
                    cl_loc[pl.ds(ncg + t * 16, 16)] = jnp.zeros((16,), I32)
                nblk = (ncg + (G - 1)) // G

                def gbody(j, _):
                    for q in range(G // 16):
                        st_cid[pl.ds(q * 16, 16)] = cl_cid[pl.ds(j * G + q * 16, 16)]
                        st_loc[pl.ds(q * 16, 16)] = cl_loc[pl.ds(j * G + q * 16, 16)]
                    pltpu.async_copy(xloc_h.at[st_loc], gbuf, sem).wait()
                    pltpu.sync_copy(gbuf, sums_sh.at[st_cid], add=True)
                    pltpu.sync_copy(ones_b, cnt_sh.at[st_cid], add=True)
                    return 0

                lax.fori_loop(0, nblk, gbody, 0)
                plsc.subcore_barrier()

                mbase = jnp.minimum(sid * MR, CS - MR)
                pltpu.sync_copy(cnt_sh, cnt_t)

                for (o, n) in ((0, 128), (128, 128), (256, 128), (384, MR - 384)):
                    pltpu.sync_copy(sums_sh.at[pl.ds(mbase + o, n), :],
                                    gbuf.at[pl.ds(0, n)])

                    def rbody(r, _):
                        c = cnt_t[mbase + o + r]
                        sc = 1.0 / jnp.maximum(jnp.full((16,), c, F32), 1.0)
                        inc = (mbase + o + r) >= sid * MR
                        incf = jnp.where(jnp.full((16,), inc, jnp.bool_), 1.0, 0.0)
                        for w in range(12):
                            v = gbuf[r, pl.ds(w * 16, 16)] * sc
                            gbuf[r, pl.ds(w * 16, 16)] = v
                            acc[pl.ds(w * 16, 16)] = acc[pl.ds(w * 16, 16)] + v * incf
                        v = gbuf[r, pl.ds(D - 16, 16)] * sc
                        gbuf[r, pl.ds(D - 16, 16)] = v
                        tmf = jnp.where(iota >= 8, 1.0, 0.0) * incf
                        acc[pl.ds(D - 16, 16)] = acc[pl.ds(D - 16, 16)] + v * tmf
                        return 0

                    lax.fori_loop(0, n, rbody, 0)
                    pltpu.sync_copy(
                        gbuf.at[pl.ds(0, n)],
                        out_h.at[b, pl.ds(chunk_base + mbase + o, n), :])

                def ebody(q, carry):
                    first, elen = carry
                    rpos = q * 16 + iota
                    row = mbase + rpos
                    cvv = cnt_t[pl.ds(mbase + q * 16, 16)]
                    m = (cvv == 0.0) & (rpos < MR) & (row >= sid * MR)
                    gids = chunk_base + row
                    plsc.store_compressed(elist.at[pl.ds(elen, 16)], gids, mask=m)
                    fcand = jnp.min(jnp.where(m, gids, BIGI))
                    return (jnp.minimum(first, fcand),
                            elen + jnp.sum(m.astype(I32)))

                first, elen = lax.fori_loop(
                    0, (MR + 15) // 16, ebody, (BIGI, jnp.int32(0)))

                def pbody(q, _):
                    pos = q * 16 + iota
                    v = elist[pl.ds(q * 16, 16)]
                    elist[pl.ds(q * 16, 16)] = jnp.where(
                        pos >= elen, jnp.full((16,), first, I32), v)
                    return 0

                lax.fori_loop(0, SLOT // 16, pbody, 0)
                pltpu.sync_copy(elist.at[pl.ds(0, SLOT)],
                                slots_h.at[b, cid, k, sid])
                lbuf[pl.ds(0, 16)] = jnp.full((16,), elen, I32)
                pltpu.sync_copy(lbuf.at[pl.ds(0, 8)],
                                lens_h.at[b, cid, k, sid])

            pltpu.sync_copy(acc, parts_h.at[b, cid, sid])

    return k2(xloc, uidx, lidx)


def _run_k3(out_ref, parts, lens, slots):
    @pl.kernel(
        out_type=(),
        mesh=_mesh(),
        scratch_types=[
            pltpu.VMEM((2, 16, 208), F32),
            pltpu.VMEM((2, NCH, 16, 8), I32),
            pltpu.VMEM((SLOT,), I32),
            pltpu.VMEM((80,), I32),
            pltpu.VMEM((208,), F32),
            pltpu.VMEM((80, D), F32),
            pltpu.SemaphoreType.DMA,
        ],
    )
    def k3(out_h, parts_h, lens_h, slots_h,
           pbuf, lbufv, sbuf, stage, avrow, ablk, sem):
        cid = lax.axis_index("c")
        sid = lax.axis_index("s")
        for b in range(BB):
            pltpu.sync_copy(parts_h.at[b], pbuf)
            pltpu.sync_copy(lens_h.at[b], lbufv)
            tot = jnp.int32(0)
            for c2 in range(2):
                for k2 in range(NCH):
                    def sb(s2, t):
                        return t + lbufv[c2, k2, s2, 0]
                    tot = lax.fori_loop(0, 16, sb, tot)
            nw = jnp.maximum((NUSER - tot).astype(F32), 1.0)
            inv = 1.0 / jnp.full((16,), nw, F32)
            for w in range(13):
                def rb(i, v):
                    return (v + pbuf[0, i, pl.ds(w * 16, 16)]
                            + pbuf[1, i, pl.ds(w * 16, 16)])
                s = lax.fori_loop(0, 16, rb, jnp.zeros((16,), F32))
                avrow[pl.ds(w * 16, 16)] = s * inv

            def ab(r, _):
                for w in range(12):
                    ablk[r, pl.ds(w * 16, 16)] = avrow[pl.ds(w * 16, 16)]
                ablk[r, pl.ds(D - 16, 16)] = avrow[pl.ds(D - 16, 16)]
                return 0

            lax.fori_loop(0, 80, ab, 0)

            for k2 in range(NCH):
                ln = lbufv[cid, k2, sid, 0]
                pltpu.sync_copy(slots_h.at[b, cid, k2, sid], sbuf)
                trips = (ln + 79) // 80

                def tb(t, _):
                    for q in range(5):
                        stage[pl.ds(q * 16, 16)] = sbuf[pl.ds(t * 80 + q * 16, 16)]
                    pltpu.async_copy(ablk, out_h.at[b].at[stage], sem).wait()
                    return 0

                lax.fori_loop(0, trips, tb, 0)

    k3(out_ref, parts, lens, slots)


def kernel(x_location, x_mobility_batch, x_text_batch, sorted_user, sorted_location):
    ul = jnp.concatenate(
        [x_mobility_batch[:, 0, :, 0], x_text_batch[:, 0, :, 0]], axis=1
    ).astype(I32)
    ll = jnp.concatenate(
        [x_mobility_batch[:, 0, :, 1], x_text_batch[:, 0, :, 1]], axis=1
    ).astype(I32)
    ul = jnp.pad(ul, ((0, 0), (0, EP - E2)))
    ll = jnp.pad(ll, ((0, 0), (0, EP - E2)))
    uidx, lidx = _run_k1(ul, ll, sorted_user.astype(I32), sorted_location.astype(I32))
    out, slots, lens, parts = _run_k2(x_location.astype(F32), uidx, lidx)
    o_ref = jax.new_ref(out)
    _run_k3(o_ref, parts, lens, slots)
    return o_ref[...]


# speedup vs baseline: 1.8027x; 1.8027x over previous
import jax
import jax.numpy as jnp
from jax import lax
from jax.experimental import pallas as pl
from jax.experimental.pallas import tpu as pltpu
from jax.experimental.pallas import tpu_sc as plsc

NUSER = 50000
NLOC = 50000
BB = 4
E2 = 100000
D = 200
EP = 100352
TPT1 = EP // 32
SLICE2 = EP // 16
NV2 = SLICE2 // 16
HALF = NUSER // 2
NCH = 4
CS = HALF // NCH
CSP = 6272
DUMMY = 6256
RZ = CSP // 16
MR = 391
G = 128
CLCAP = SLICE2 + G
SLOT = 400
ELCAP = 416
BIG = 1 << 30
F32 = jnp.float32
I32 = jnp.int32


def _mesh():
    return plsc.VectorSubcoreMesh(
        core_axis_name="c", subcore_axis_name="s", num_cores=2, num_subcores=16
    )


def _iota16():
    return lax.iota(I32, 16)


def _run_k1(ul, ll, su, sl):
    @pl.kernel(
        out_type=jax.ShapeDtypeStruct((BB, EP), I32),
        mesh=_mesh(),
        compiler_params=pltpu.CompilerParams(use_tc_tiling_on_sc=False, needs_layout_passes=False),
        scratch_types=[
            pltpu.VMEM((NUSER,), I32),
            pltpu.VMEM((NLOC,), I32),
            pltpu.VMEM((TPT1,), I32),
            pltpu.VMEM((TPT1,), I32),
            pltpu.VMEM((TPT1,), I32),
        ],
    )
    def k1(ul_h, ll_h, su_h, sl_h, pk_h, tblu, tbll, ibu, ibl, ob):
        cid = lax.axis_index("c")
        sid = lax.axis_index("s")
        wid = sid * 2 + cid
        base_e = wid * TPT1
        iota = _iota16()
        pltpu.sync_copy(su_h, tblu)
        pltpu.sync_copy(sl_h, tbll)
        for b in range(BB):
            pltpu.sync_copy(ul_h.at[b, pl.ds(base_e, TPT1)], ibu)
            pltpu.sync_copy(ll_h.at[b, pl.ds(base_e, TPT1)], ibl)

            def body(i, _):
                uv = plsc.load_gather(tblu, [ibu[pl.ds(i * 16, 16)]])
                lv = plsc.load_gather(tbll, [ibl[pl.ds(i * 16, 16)]])
                pk = (uv << 16) | lv
                gpos = base_e + i * 16 + iota
                ob[pl.ds(i * 16, 16)] = jnp.where(
                    gpos < E2, pk, jnp.int32(-65536))
                return 0

            lax.fori_loop(0, TPT1 // 16, body, 0)
            pltpu.sync_copy(ob, pk_h.at[b, pl.ds(base_e, TPT1)])

    return k1(ul, ll, su, sl)


def _run_k2(xloc, epk):
    @pl.kernel(
        out_type=(
            jax.ShapeDtypeStruct((BB, NUSER, D), F32),
            jax.ShapeDtypeStruct((BB, 2, NCH, 16, SLOT), I32),
            jax.ShapeDtypeStruct((BB, 2, NCH, 16, 16), I32),
            jax.ShapeDtypeStruct((BB, 2, 16, 208), F32),
        ),
        mesh=_mesh(),
        compiler_params=pltpu.CompilerParams(use_tc_tiling_on_sc=False, needs_layout_passes=False),
        scratch_types=[
            pltpu.VMEM((SLICE2,), I32),
            pltpu.VMEM((CLCAP,), I32),
            pltpu.VMEM((G,), I32),
            pltpu.VMEM((G,), I32),
            pltpu.VMEM((G, D), F32),
            pltpu.VMEM((G,), F32),
            pltpu.VMEM((16, D), F32),
            pltpu.VMEM((400,), F32),
            pltpu.VMEM((CSP,), F32),
            pltpu.VMEM((ELCAP,), I32),
            pltpu.VMEM((16,), I32),
            pltpu.VMEM((208,), F32),
            pltpu.VMEM_SHARED((CSP, D), F32),
            pltpu.VMEM_SHARED((CSP,), F32),
            pltpu.SemaphoreType.DMA,
        ],
    )
    def k2(xloc_h, epk_h, out_h, slots_h, lens_h, parts_h,
           e_sl, cl, st_cid, st_loc, gbuf, ones_b,
           zrow2, zrow1, cnt_t, elist, lbuf, acc, sums_sh, cnt_sh, sem):
        cid = lax.axis_index("c")
        sid = lax.axis_index("s")
        iota = _iota16()
        half_base = cid * HALF
        zero16 = jnp.zeros((16,), F32)

        for r in range(16):
            for w in range(12):
                zrow2[r, pl.ds(w * 16, 16)] = zero16
            zrow2[r, pl.ds(D - 16, 16)] = zero16
        for q in range(G // 16):
            ones_b[pl.ds(q * 16, 16)] = jnp.full((16,), 1.0, F32)
        for q in range(400 // 16):
            zrow1[pl.ds(q * 16, 16)] = zero16

        def chunk_step(t, _):
            b = t // NCH
            k = t - b * NCH
            chunk_base = half_base + k * CS

            @pl.when(k == 0)
            def _():
                pltpu.sync_copy(epk_h.at[b, pl.ds(sid * SLICE2, SLICE2)], e_sl)
                for w in range(13):
                    acc[pl.ds(w * 16, 16)] = zero16

            plsc.subcore_barrier()
            rz = sid * RZ
            for j in range(RZ // 16):
                pltpu.sync_copy(zrow2, sums_sh.at[pl.ds(rz + j * 16, 16)])
            if RZ % 16:
                pltpu.sync_copy(zrow2.at[pl.ds(0, RZ % 16)],
                                sums_sh.at[pl.ds(rz + (RZ // 16) * 16, RZ % 16)])
            pltpu.sync_copy(zrow1.at[pl.ds(0, RZ)], cnt_sh.at[pl.ds(rz, RZ)])
            plsc.subcore_barrier()

            def cbody(i, off):
                v = e_sl[pl.ds(i * 16, 16)]
                uv = lax.shift_right_logical(v, 16)
                cv = uv - chunk_base
                m = (cv >= 0) & (cv < CS)
                pk2 = (cv << 16) | (v & 0xFFFF)
                plsc.store_compressed(cl.at[pl.ds(off, 16)], pk2, mask=m)
                return off + jnp.sum(m.astype(I32))

            ncg = lax.fori_loop(0, NV2, cbody, jnp.int32(0))
            for t2 in range(G // 16):
                cl[pl.ds(ncg + t2 * 16, 16)] = jnp.full((16,), DUMMY << 16, I32)
            nblk = (ncg + (G - 1)) // G

            def gbody(j, _):
                for q in range(G // 16):
                    v = cl[pl.ds(j * G + q * 16, 16)]
                    st_cid[pl.ds(q * 16, 16)] = lax.shift_right_logical(v, 16)
                    st_loc[pl.ds(q * 16, 16)] = v & 0xFFFF
                pltpu.async_copy(xloc_h.at[st_loc], gbuf, sem).wait()
                pltpu.sync_copy(gbuf, sums_sh.at[st_cid], add=True)
                pltpu.sync_copy(ones_b, cnt_sh.at[st_cid], add=True)
                return 0

            lax.fori_loop(0, nblk, gbody, 0)
            plsc.subcore_barrier()

            mbase = jnp.minimum(sid * MR, CS - MR)
            pltpu.sync_copy(cnt_sh, cnt_t)

            for (o, n) in ((0, 128), (128, 128), (256, 128), (384, MR - 384)):
                pltpu.sync_copy(sums_sh.at[pl.ds(mbase + o, n), :],
                                gbuf.at[pl.ds(0, n)])

                def rbody(r, _):
                    c = cnt_t[pl.ds(mbase + o + r, 16)][0]
                    sc = 1.0 / jnp.maximum(jnp.full((16,), c, F32), 1.0)
                    inc = (mbase + o + r) >= sid * MR
                    incf = jnp.where(jnp.full((16,), inc, jnp.bool_), 1.0, 0.0)
                    vt = gbuf[r, pl.ds(D - 16, 16)]
                    for w in range(12):
                        v = gbuf[r, pl.ds(w * 16, 16)] * sc
                        gbuf[r, pl.ds(w * 16, 16)] = v
                        acc[pl.ds(w * 16, 16)] = acc[pl.ds(w * 16, 16)] + v * incf
                    v = vt * sc
                    gbuf[r, pl.ds(D - 16, 16)] = v
                    tmf = jnp.where(iota >= 8, 1.0, 0.0) * incf
                    acc[pl.ds(D - 16, 16)] = acc[pl.ds(D - 16, 16)] + v * tmf
                    return 0

                lax.fori_loop(0, n, rbody, 0)
                pltpu.sync_copy(
                    gbuf.at[pl.ds(0, n)],
                    out_h.at[b, pl.ds(chunk_base + mbase + o, n), :])

            def ebody(q, carry):
                first, elen = carry
                rpos = q * 16 + iota
                row = mbase + rpos
                cvv = cnt_t[pl.ds(mbase + q * 16, 16)]
                m = (cvv == 0.0) & (rpos < MR) & (row >= sid * MR)
                gids = chunk_base + row
                plsc.store_compressed(elist.at[pl.ds(elen, 16)], gids, mask=m)
                fcand = jnp.min(jnp.where(m, gids, jnp.int32(BIG)))
                return (jnp.minimum(first, fcand),
                        elen + jnp.sum(m.astype(I32)))

            first, elen = lax.fori_loop(
                0, (MR + 15) // 16, ebody, (jnp.int32(BIG), jnp.int32(0)))

            def pbody(q, _):
                pos = q * 16 + iota
                v = elist[pl.ds(q * 16, 16)]
                elist[pl.ds(q * 16, 16)] = jnp.where(
                    pos >= elen, jnp.full((16,), first, I32), v)
                return 0

            lax.fori_loop(0, SLOT // 16, pbody, 0)
            pltpu.sync_copy(elist.at[pl.ds(0, SLOT)],
                            slots_h.at[b, cid, k, sid])
            lbuf[pl.ds(0, 16)] = jnp.full((16,), elen, I32)
            pltpu.sync_copy(lbuf, lens_h.at[b, cid, k, sid])

            @pl.when(k == NCH - 1)
            def _():
                pltpu.sync_copy(acc, parts_h.at[b, cid, sid])

            return 0

        lax.fori_loop(0, BB * NCH, chunk_step, 0)

    return k2(xloc, epk)


def _run_k3(out_ref, parts, lens, slots):
    @pl.kernel(
        out_type=(),
        mesh=_mesh(),
        compiler_params=pltpu.CompilerParams(use_tc_tiling_on_sc=False, needs_layout_passes=False),
        scratch_types=[
            pltpu.VMEM((2, 16, 208), F32),
            pltpu.VMEM((2, NCH, 16, 16), I32),
            pltpu.VMEM((SLOT,), I32),
            pltpu.VMEM((80,), I32),
            pltpu.VMEM((208,), F32),
            pltpu.VMEM((80, D), F32),
            pltpu.SemaphoreType.DMA,
        ],
    )
    def k3(out_h, parts_h, lens_h, slots_h,
           pbuf, lbufv, sbuf, stage, avrow, ablk, sem):
        cid = lax.axis_index("c")
        sid = lax.axis_index("s")
        for b in range(BB):
            pltpu.sync_copy(parts_h.at[b], pbuf)
            pltpu.sync_copy(lens_h.at[b], lbufv)
            tot = jnp.int32(0)
            for c2 in range(2):
                for k2 in range(NCH):
                    def sb(s2, t):
                        return t + lbufv[c2, k2, s2][0]
                    tot = lax.fori_loop(0, 16, sb, tot)
            nw = jnp.maximum((NUSER - tot).astype(F32), 1.0)
            inv = 1.0 / jnp.full((16,), nw, F32)
            for w in range(13):
                def rb(i, v):
                    return (v + pbuf[0, i, pl.ds(w * 16, 16)]
                            + pbuf[1, i, pl.ds(w * 16, 16)])
                s = lax.fori_loop(0, 16, rb, jnp.zeros((16,), F32))
                avrow[pl.ds(w * 16, 16)] = s * inv

            def ab(r, _):
                for w in range(12):
                    ablk[r, pl.ds(w * 16, 16)] = avrow[pl.ds(w * 16, 16)]
                ablk[r, pl.ds(D - 16, 16)] = avrow[pl.ds(D - 16, 16)]
                return 0

            lax.fori_loop(0, 80, ab, 0)

            for k2 in range(NCH):
                ln = lbufv[cid, k2, sid][0]
                pltpu.sync_copy(slots_h.at[b, cid, k2, sid], sbuf)
                trips = (ln + 79) // 80

                def tb(t, _):
                    for q in range(5):
                        stage[pl.ds(q * 16, 16)] = sbuf[pl.ds(t * 80 + q * 16, 16)]
                    pltpu.async_copy(ablk, out_h.at[b].at[stage], sem).wait()
                    return 0

                lax.fori_loop(0, trips, tb, 0)

    k3(out_ref, parts, lens, slots)


def kernel(x_location, x_mobility_batch, x_text_batch, sorted_user, sorted_location):
    ul = jnp.concatenate(
        [x_mobility_batch[:, 0, :, 0], x_text_batch[:, 0, :, 0]], axis=1
    ).astype(I32)
    ll = jnp.concatenate(
        [x_mobility_batch[:, 0, :, 1], x_text_batch[:, 0, :, 1]], axis=1
    ).astype(I32)
    ul = jnp.pad(ul, ((0, 0), (0, EP - E2)))
    ll = jnp.pad(ll, ((0, 0), (0, EP - E2)))
    epk = _run_k1(ul, ll, sorted_user.astype(I32), sorted_location.astype(I32))
    out, slots, lens, parts = _run_k2(x_location.astype(F32), epk)
    o_ref = jax.new_ref(out)
    _run_k3(o_ref, parts, lens, slots)
    return o_ref[...]
